# baseline (speedup 1.0000x reference)
import jax
import jax.numpy as jnp
from jax.experimental import pallas as pl
from jax.experimental.pallas import tpu as pltpu

N = 8192
M = 8192
K = 128
BI = 1024
BJ = 1024
NI = N // BI
NJ = M // BJ
TOTAL = NI * NJ
KA = K + 2
LANE = 128
SUB = 16


def _aug_x(b):
    g = 0.5 * jnp.sum(b * b, axis=1, keepdims=True)
    neg1 = jnp.full_like(g, -1.0)
    return jnp.concatenate([b, neg1, g], axis=1).astype(jnp.bfloat16)


def _aug_y(b):
    h = 0.5 * jnp.sum(b * b, axis=1, keepdims=True)
    neg1 = jnp.full_like(h, -1.0)
    return jnp.concatenate([b, h, neg1], axis=1).astype(jnp.bfloat16)


def _sweep(p):
    pc = None
    prs = []
    for r in range(BI // SUB):
        row = p[r * SUB:(r + 1) * SUB, :]
        pc = row if pc is None else jnp.maximum(pc, row)
        t = row[:, 0:LANE]
        for c in range(1, BJ // LANE):
            t = jnp.maximum(t, row[:, c * LANE:(c + 1) * LANE])
        prs.append(t)
    pr = jnp.concatenate(prs, axis=0)
    return pr, pc


def _chamfer_tile(x_ref, y_ref, out_ref, xa_s, ya_s, pa, pb, rowacc, colacc):
    i = pl.program_id(0)
    j = pl.program_id(1)
    t = i * NJ + j
    par = jax.lax.rem(t, 2)

    @pl.when(j == 0)
    def _():
        xa_s[...] = _aug_x(x_ref[...])

    @pl.when(i == 0)
    def _():
        ya_s[j] = _aug_y(y_ref[...])

    def _dot():
        return jax.lax.dot_general(
            xa_s[...], ya_s[j], (((1,), (1,)), ((), ())),
            preferred_element_type=jnp.float32,
        ).astype(jnp.bfloat16)

    @pl.when(par == 0)
    def _():
        pa[...] = _dot()

    @pl.when(par == 1)
    def _():
        pb[...] = _dot()

    def _accumulate(src_ref, tt):
        i1 = tt // NJ
        j1 = jax.lax.rem(tt, NJ)
        pr, pc = _sweep(src_ref[...])

        @pl.when(j1 == 0)
        def _():
            rowacc[i1] = pr

        @pl.when(j1 > 0)
        def _():
            rowacc[i1] = jnp.maximum(rowacc[i1], pr)

        @pl.when(i1 == 0)
        def _():
            colacc[j1] = pc

        @pl.when(i1 > 0)
        def _():
            colacc[j1] = jnp.maximum(colacc[j1], pc)

    @pl.when((t > 0) & (par == 1))
    def _():
        _accumulate(pa, t - 1)

    @pl.when((t > 0) & (par == 0))
    def _():
        _accumulate(pb, t - 1)

    @pl.when(t == TOTAL - 1)
    def _():
        _accumulate(pb, t)
        rm = jnp.max(rowacc[...], axis=2).astype(jnp.float32)
        d_xy = jnp.maximum(-2.0 * rm, 0.0)
        cm = jnp.max(colacc[...], axis=1).astype(jnp.float32)
        d_yx = jnp.maximum(-2.0 * cm, 0.0)
        out_ref[...] = (jnp.sum(d_xy, keepdims=True)
                        + jnp.sum(d_yx, keepdims=True))


def kernel(x, y):
    out = pl.pallas_call(
        _chamfer_tile,
        grid=(NI, NJ),
        in_specs=[
            pl.BlockSpec((BI, K), lambda i, j: (i, 0)),
            pl.BlockSpec((BJ, K), lambda i, j: (j, 0)),
        ],
        out_specs=pl.BlockSpec((1, 1), lambda i, j: (0, 0)),
        out_shape=jax.ShapeDtypeStruct((1, 1), jnp.float32),
        scratch_shapes=[
            pltpu.VMEM((BI, KA), jnp.bfloat16),
            pltpu.VMEM((NJ, BJ, KA), jnp.bfloat16),
            pltpu.VMEM((BI, BJ), jnp.bfloat16),
            pltpu.VMEM((BI, BJ), jnp.bfloat16),
            pltpu.VMEM((NI, BI, LANE), jnp.bfloat16),
            pltpu.VMEM((NJ, SUB, BJ), jnp.bfloat16),
        ],
        compiler_params=pltpu.CompilerParams(
            dimension_semantics=("arbitrary", "arbitrary"),
        ),
    )(x, y)
    return out[0, 0]


# speedup vs baseline: 1.1772x; 1.1772x over previous
import jax
import jax.numpy as jnp
from jax.experimental import pallas as pl
from jax.experimental.pallas import tpu as pltpu

N = 8192
M = 8192
K = 128
BI = 1024
BJ = 1024
NI = N // BI
NJ = M // BJ
KA = K + 2
LANE = 128
SUB = 8
CH = 4
CW = BJ // CH


def _aug_x(b):
    g = 0.5 * jnp.sum(b * b, axis=1, keepdims=True)
    neg1 = jnp.full_like(g, -1.0)
    return jnp.concatenate([b, neg1, g], axis=1).astype(jnp.bfloat16)


def _aug_y(b):
    h = 0.5 * jnp.sum(b * b, axis=1, keepdims=True)
    neg1 = jnp.full_like(h, -1.0)
    return jnp.concatenate([b, h, neg1], axis=1).astype(jnp.bfloat16)


def _chamfer_tile(x_ref, y_ref, out_ref, xa_s, ya_s, rowacc, colacc):
    i = pl.program_id(0)
    j = pl.program_id(1)

    @pl.when(j == 0)
    def _():
        xa_s[...] = _aug_x(x_ref[...])

    @pl.when(i == 0)
    def _():
        ya_s[j] = _aug_y(y_ref[...])

    xa = xa_s[...]
    ya = ya_s[j]

    pr = None
    pcs = []
    for s in range(CH):
        ps = jax.lax.dot_general(
            xa, ya[s * CW:(s + 1) * CW, :], (((1,), (1,)), ((), ())),
            preferred_element_type=jnp.float32,
        )
        t = ps[:, 0:LANE]
        for c in range(1, CW // LANE):
            t = jnp.maximum(t, ps[:, c * LANE:(c + 1) * LANE])
        pr = t if pr is None else jnp.maximum(pr, t)
        pc_s = ps[0:SUB, :]
        for r in range(1, BI // SUB):
            pc_s = jnp.maximum(pc_s, ps[r * SUB:(r + 1) * SUB, :])
        pcs.append(pc_s)
    pc = jnp.concatenate(pcs, axis=1)

    @pl.when(j == 0)
    def _():
        rowacc[i] = pr

    @pl.when(j > 0)
    def _():
        rowacc[i] = jnp.maximum(rowacc[i], pr)

    @pl.when(i == 0)
    def _():
        colacc[j] = pc

    @pl.when(i > 0)
    def _():
        colacc[j] = jnp.maximum(colacc[j], pc)

    @pl.when((i == NI - 1) & (j == NJ - 1))
    def _():
        rm = jnp.max(rowacc[...], axis=2)
        d_xy = jnp.maximum(-2.0 * rm, 0.0)
        cm = jnp.max(colacc[...], axis=1)
        d_yx = jnp.maximum(-2.0 * cm, 0.0)
        out_ref[...] = (jnp.sum(d_xy, keepdims=True)
                        + jnp.sum(d_yx, keepdims=True))


def kernel(x, y):
    out = pl.pallas_call(
        _chamfer_tile,
        grid=(NI, NJ),
        in_specs=[
            pl.BlockSpec((BI, K), lambda i, j: (i, 0)),
            pl.BlockSpec((BJ, K), lambda i, j: (j, 0)),
        ],
        out_specs=pl.BlockSpec((1, 1), lambda i, j: (0, 0)),
        out_shape=jax.ShapeDtypeStruct((1, 1), jnp.float32),
        scratch_shapes=[
            pltpu.VMEM((BI, KA), jnp.bfloat16),
            pltpu.VMEM((NJ, BJ, KA), jnp.bfloat16),
            pltpu.VMEM((NI, BI, LANE), jnp.float32),
            pltpu.VMEM((NJ, SUB, BJ), jnp.float32),
        ],
        compiler_params=pltpu.CompilerParams(
            dimension_semantics=("arbitrary", "arbitrary"),
        ),
    )(x, y)
    return out[0, 0]


# speedup vs baseline: 1.5242x; 1.2948x over previous
import jax
import jax.numpy as jnp
from jax.experimental import pallas as pl
from jax.experimental.pallas import tpu as pltpu

N = 8192
M = 8192
K = 128
BI = 2048
BJ = 1024
NI = N // BI
NJ = M // BJ
KA = K + 2
LANE = 128
SUB = 8
CH = 4
CW = BJ // CH


def _aug_x(b):
    g = 0.5 * jnp.sum(b * b, axis=1, keepdims=True)
    neg1 = jnp.full_like(g, -1.0)
    return jnp.concatenate([b, neg1, g], axis=1).astype(jnp.bfloat16)


def _aug_y(b):
    h = 0.5 * jnp.sum(b * b, axis=1, keepdims=True)
    neg1 = jnp.full_like(h, -1.0)
    return jnp.concatenate([b, h, neg1], axis=1).astype(jnp.bfloat16)


def _chamfer_tile(x_ref, y_ref, out_ref, xa_s, ya_s, rowacc, colacc):
    i = pl.program_id(0)
    j = pl.program_id(1)

    @pl.when(j == 0)
    def _():
        xa_s[...] = _aug_x(x_ref[...])

    @pl.when(i == 0)
    def _():
        ya_s[j] = _aug_y(y_ref[...])

    xa = xa_s[...]
    ya = ya_s[j]

    pr = None
    pcs = []
    for s in range(CH):
        ps = jax.lax.dot_general(
            xa, ya[s * CW:(s + 1) * CW, :], (((1,), (1,)), ((), ())),
            preferred_element_type=jnp.float32,
        )
        t = ps[:, 0:LANE]
        for c in range(1, CW // LANE):
            t = jnp.maximum(t, ps[:, c * LANE:(c + 1) * LANE])
        pr = t if pr is None else jnp.maximum(pr, t)
        pc_s = ps[0:SUB, :]
        for r in range(1, BI // SUB):
            pc_s = jnp.maximum(pc_s, ps[r * SUB:(r + 1) * SUB, :])
        pcs.append(pc_s)
    pc = jnp.concatenate(pcs, axis=1)

    @pl.when(j == 0)
    def _():
        rowacc[i] = pr

    @pl.when(j > 0)
    def _():
        rowacc[i] = jnp.maximum(rowacc[i], pr)

    @pl.when(i == 0)
    def _():
        colacc[j] = pc

    @pl.when(i > 0)
    def _():
        colacc[j] = jnp.maximum(colacc[j], pc)

    @pl.when((i == NI - 1) & (j == NJ - 1))
    def _():
        rm = jnp.max(rowacc[...], axis=2)
        d_xy = jnp.maximum(-2.0 * rm, 0.0)
        cm = jnp.max(colacc[...], axis=1)
        d_yx = jnp.maximum(-2.0 * cm, 0.0)
        out_ref[...] = (jnp.sum(d_xy, keepdims=True)
                        + jnp.sum(d_yx, keepdims=True))


def kernel(x, y):
    out = pl.pallas_call(
        _chamfer_tile,
        grid=(NI, NJ),
        in_specs=[
            pl.BlockSpec((BI, K), lambda i, j: (i, 0)),
            pl.BlockSpec((BJ, K), lambda i, j: (j, 0)),
        ],
        out_specs=pl.BlockSpec((1, 1), lambda i, j: (0, 0)),
        out_shape=jax.ShapeDtypeStruct((1, 1), jnp.float32),
        scratch_shapes=[
            pltpu.VMEM((BI, KA), jnp.bfloat16),
            pltpu.VMEM((NJ, BJ, KA), jnp.bfloat16),
            pltpu.VMEM((NI, BI, LANE), jnp.float32),
            pltpu.VMEM((NJ, SUB, BJ), jnp.float32),
        ],
        compiler_params=pltpu.CompilerParams(
            dimension_semantics=("arbitrary", "arbitrary"),
        ),
    )(x, y)
    return out[0, 0]


# speedup vs baseline: 1.6489x; 1.0818x over previous
import jax
import jax.numpy as jnp
from jax.experimental import pallas as pl
from jax.experimental.pallas import tpu as pltpu

N = 8192
M = 8192
K = 128
BI = 4096
BJ = 1024
NI = N // BI
NJ = M // BJ
KA = K + 2
LANE = 128
SUB = 8
CH = 4
CW = BJ // CH


def _aug_x(b):
    g = 0.5 * jnp.sum(b * b, axis=1, keepdims=True)
    neg1 = jnp.full_like(g, -1.0)
    return jnp.concatenate([b, neg1, g], axis=1).astype(jnp.bfloat16)


def _aug_y(b):
    h = 0.5 * jnp.sum(b * b, axis=1, keepdims=True)
    neg1 = jnp.full_like(h, -1.0)
    return jnp.concatenate([b, h, neg1], axis=1).astype(jnp.bfloat16)


def _chamfer_tile(x_ref, y_ref, out_ref, xa_s, ya_s, rowacc, colacc):
    i = pl.program_id(0)
    j = pl.program_id(1)

    @pl.when(j == 0)
    def _():
        xa_s[...] = _aug_x(x_ref[...])

    @pl.when(i == 0)
    def _():
        ya_s[j] = _aug_y(y_ref[...])

    xa = xa_s[...]
    ya = ya_s[j]

    pr = None
    pcs = []
    for s in range(CH):
        ps = jax.lax.dot_general(
            xa, ya[s * CW:(s + 1) * CW, :], (((1,), (1,)), ((), ())),
            preferred_element_type=jnp.float32,
        )
        t = ps[:, 0:LANE]
        for c in range(1, CW // LANE):
            t = jnp.maximum(t, ps[:, c * LANE:(c + 1) * LANE])
        pr = t if pr is None else jnp.maximum(pr, t)
        pc_s = ps[0:SUB, :]
        for r in range(1, BI // SUB):
            pc_s = jnp.maximum(pc_s, ps[r * SUB:(r + 1) * SUB, :])
        pcs.append(pc_s)
    pc = jnp.concatenate(pcs, axis=1)

    @pl.when(j == 0)
    def _():
        rowacc[i] = pr

    @pl.when(j > 0)
    def _():
        rowacc[i] = jnp.maximum(rowacc[i], pr)

    @pl.when(i == 0)
    def _():
        colacc[j] = pc

    @pl.when(i > 0)
    def _():
        colacc[j] = jnp.maximum(colacc[j], pc)

    @pl.when((i == NI - 1) & (j == NJ - 1))
    def _():
        rm = jnp.max(rowacc[...], axis=2)
        d_xy = jnp.maximum(-2.0 * rm, 0.0)
        cm = jnp.max(colacc[...], axis=1)
        d_yx = jnp.maximum(-2.0 * cm, 0.0)
        out_ref[...] = (jnp.sum(d_xy, keepdims=True)
                        + jnp.sum(d_yx, keepdims=True))


def kernel(x, y):
    out = pl.pallas_call(
        _chamfer_tile,
        grid=(NI, NJ),
        in_specs=[
            pl.BlockSpec((BI, K), lambda i, j: (i, 0)),
            pl.BlockSpec((BJ, K), lambda i, j: (j, 0)),
        ],
        out_specs=pl.BlockSpec((1, 1), lambda i, j: (0, 0)),
        out_shape=jax.ShapeDtypeStruct((1, 1), jnp.float32),
        scratch_shapes=[
            pltpu.VMEM((BI, KA), jnp.bfloat16),
            pltpu.VMEM((NJ, BJ, KA), jnp.bfloat16),
            pltpu.VMEM((NI, BI, LANE), jnp.float32),
            pltpu.VMEM((NJ, SUB, BJ), jnp.float32),
        ],
        compiler_params=pltpu.CompilerParams(
            dimension_semantics=("arbitrary", "arbitrary"),
        ),
    )(x, y)
    return out[0, 0]


# speedup vs baseline: 1.7585x; 1.0665x over previous
import jax
import jax.numpy as jnp
from jax.experimental import pallas as pl
from jax.experimental.pallas import tpu as pltpu

N = 8192
M = 8192
K = 128
BI = 8192
BJ = 1024
NI = N // BI
NJ = M // BJ
KA = K + 2
LANE = 128
SUB = 8
CH = 4
CW = BJ // CH


def _aug_x(b):
    g = 0.5 * jnp.sum(b * b, axis=1, keepdims=True)
    neg1 = jnp.full_like(g, -1.0)
    return jnp.concatenate([b, neg1, g], axis=1).astype(jnp.bfloat16)


def _aug_y(b):
    h = 0.5 * jnp.sum(b * b, axis=1, keepdims=True)
    neg1 = jnp.full_like(h, -1.0)
    return jnp.concatenate([b, h, neg1], axis=1).astype(jnp.bfloat16)


def _chamfer_tile(x_ref, y_ref, out_ref, xa_s, ya_s, rowacc, colacc):
    i = pl.program_id(0)
    j = pl.program_id(1)

    @pl.when(j == 0)
    def _():
        xa_s[...] = _aug_x(x_ref[...])

    @pl.when(i == 0)
    def _():
        ya_s[j] = _aug_y(y_ref[...])

    xa = xa_s[...]
    ya = ya_s[j]

    pr = None
    pcs = []
    for s in range(CH):
        ps = jax.lax.dot_general(
            xa, ya[s * CW:(s + 1) * CW, :], (((1,), (1,)), ((), ())),
            preferred_element_type=jnp.float32,
        )
        t = ps[:, 0:LANE]
        for c in range(1, CW // LANE):
            t = jnp.maximum(t, ps[:, c * LANE:(c + 1) * LANE])
        pr = t if pr is None else jnp.maximum(pr, t)
        pc_s = ps[0:SUB, :]
        for r in range(1, BI // SUB):
            pc_s = jnp.maximum(pc_s, ps[r * SUB:(r + 1) * SUB, :])
        pcs.append(pc_s)
    pc = jnp.concatenate(pcs, axis=1)

    @pl.when(j == 0)
    def _():
        rowacc[i] = pr

    @pl.when(j > 0)
    def _():
        rowacc[i] = jnp.maximum(rowacc[i], pr)

    @pl.when(i == 0)
    def _():
        colacc[j] = pc

    @pl.when(i > 0)
    def _():
        colacc[j] = jnp.maximum(colacc[j], pc)

    @pl.when((i == NI - 1) & (j == NJ - 1))
    def _():
        rm = jnp.max(rowacc[...], axis=2)
        d_xy = jnp.maximum(-2.0 * rm, 0.0)
        cm = jnp.max(colacc[...], axis=1)
        d_yx = jnp.maximum(-2.0 * cm, 0.0)
        out_ref[...] = (jnp.sum(d_xy, keepdims=True)
                        + jnp.sum(d_yx, keepdims=True))


def kernel(x, y):
    out = pl.pallas_call(
        _chamfer_tile,
        grid=(NI, NJ),
        in_specs=[
            pl.BlockSpec((BI, K), lambda i, j: (i, 0)),
            pl.BlockSpec((BJ, K), lambda i, j: (j, 0)),
        ],
        out_specs=pl.BlockSpec((1, 1), lambda i, j: (0, 0)),
        out_shape=jax.ShapeDtypeStruct((1, 1), jnp.float32),
        scratch_shapes=[
            pltpu.VMEM((BI, KA), jnp.bfloat16),
            pltpu.VMEM((NJ, BJ, KA), jnp.bfloat16),
            pltpu.VMEM((NI, BI, LANE), jnp.float32),
            pltpu.VMEM((NJ, SUB, BJ), jnp.float32),
        ],
        compiler_params=pltpu.CompilerParams(
            dimension_semantics=("arbitrary", "arbitrary"),
        ),
    )(x, y)
    return out[0, 0]


# speedup vs baseline: 1.8585x; 1.0569x over previous
import jax
import jax.numpy as jnp
from jax.experimental import pallas as pl
from jax.experimental.pallas import tpu as pltpu

N = 8192
M = 8192
K = 128
BI = 8192
BJ = 2048
NI = N // BI
NJ = M // BJ
KA = K + 2
LANE = 128
SUB = 8
CH = 8
CW = BJ // CH


def _aug_x(b):
    g = 0.5 * jnp.sum(b * b, axis=1, keepdims=True)
    neg1 = jnp.full_like(g, -1.0)
    return jnp.concatenate([b, neg1, g], axis=1).astype(jnp.bfloat16)


def _aug_y(b):
    h = 0.5 * jnp.sum(b * b, axis=1, keepdims=True)
    neg1 = jnp.full_like(h, -1.0)
    return jnp.concatenate([b, h, neg1], axis=1).astype(jnp.bfloat16)


def _chamfer_tile(x_ref, y_ref, out_ref, xa_s, ya_s, rowacc, colacc):
    i = pl.program_id(0)
    j = pl.program_id(1)

    @pl.when(j == 0)
    def _():
        xa_s[...] = _aug_x(x_ref[...])

    @pl.when(i == 0)
    def _():
        ya_s[j] = _aug_y(y_ref[...])

    xa = xa_s[...]
    ya = ya_s[j]

    pr = None
    pcs = []
    for s in range(CH):
        ps = jax.lax.dot_general(
            xa, ya[s * CW:(s + 1) * CW, :], (((1,), (1,)), ((), ())),
            preferred_element_type=jnp.float32,
        )
        t = ps[:, 0:LANE]
        for c in range(1, CW // LANE):
            t = jnp.maximum(t, ps[:, c * LANE:(c + 1) * LANE])
        pr = t if pr is None else jnp.maximum(pr, t)
        pc_s = ps[0:SUB, :]
        for r in range(1, BI // SUB):
            pc_s = jnp.maximum(pc_s, ps[r * SUB:(r + 1) * SUB, :])
        pcs.append(pc_s)
    pc = jnp.concatenate(pcs, axis=1)

    @pl.when(j == 0)
    def _():
        rowacc[i] = pr

    @pl.when(j > 0)
    def _():
        rowacc[i] = jnp.maximum(rowacc[i], pr)

    @pl.when(i == 0)
    def _():
        colacc[j] = pc

    @pl.when(i > 0)
    def _():
        colacc[j] = jnp.maximum(colacc[j], pc)

    @pl.when((i == NI - 1) & (j == NJ - 1))
    def _():
        rm = jnp.max(rowacc[...], axis=2)
        d_xy = jnp.maximum(-2.0 * rm, 0.0)
        cm = jnp.max(colacc[...], axis=1)
        d_yx = jnp.maximum(-2.0 * cm, 0.0)
        out_ref[...] = (jnp.sum(d_xy, keepdims=True)
                        + jnp.sum(d_yx, keepdims=True))


def kernel(x, y):
    out = pl.pallas_call(
        _chamfer_tile,
        grid=(NI, NJ),
        in_specs=[
            pl.BlockSpec((BI, K), lambda i, j: (i, 0)),
            pl.BlockSpec((BJ, K), lambda i, j: (j, 0)),
        ],
        out_specs=pl.BlockSpec((1, 1), lambda i, j: (0, 0)),
        out_shape=jax.ShapeDtypeStruct((1, 1), jnp.float32),
        scratch_shapes=[
            pltpu.VMEM((BI, KA), jnp.bfloat16),
            pltpu.VMEM((NJ, BJ, KA), jnp.bfloat16),
            pltpu.VMEM((NI, BI, LANE), jnp.float32),
            pltpu.VMEM((NJ, SUB, BJ), jnp.float32),
        ],
        compiler_params=pltpu.CompilerParams(
            dimension_semantics=("arbitrary", "arbitrary"),
        ),
    )(x, y)
    return out[0, 0]
